# baseline (speedup 1.0000x reference)
import jax
import jax.numpy as jnp
from jax.experimental import pallas as pl
from jax.experimental.pallas import tpu as pltpu


def _vq_block_kernel(zm_ref, emb_ref, embt_ref, zq_ref, idx_ref, loss_ref,
                     e2_ref):
    k_dim = emb_ref.shape[0]
    t_blk = zm_ref.shape[2]
    half = t_blk // 2

    @pl.when(pl.program_id(0) == 0)
    def _init():
        emb0 = emb_ref[...]
        e2_ref[...] = 0.25 * jnp.sum(emb0 * emb0, axis=1, keepdims=True)

    emb = emb_ref[...]
    embt = embt_ref[...]
    e2 = e2_ref[...]
    iota_k = jax.lax.broadcasted_iota(jnp.int16, (k_dim, half), 0)

    def _half(zb):
        scores = jnp.dot(emb, zb, preferred_element_type=jnp.float32)
        dist = e2 + scores
        idx = jnp.argmin(dist, axis=0)
        onehot = (iota_k == idx.astype(jnp.int16)[None, :]).astype(jnp.bfloat16)
        zq = jnp.dot(embt, onehot,
                     preferred_element_type=jnp.float32)
        resid = zb - zq
        part = jnp.sum(resid * resid, axis=(0, 1), keepdims=True)
        return zq, idx, part

    zq0, idx0, part0 = _half(zm_ref[0, :, :half])
    zq1, idx1, part1 = _half(zm_ref[0, :, half:])

    zq_ref[0, :, :half] = zq0
    zq_ref[0, :, half:] = zq1
    idx_ref[0, 0, :half] = idx0
    idx_ref[0, 0, half:] = idx1
    loss_ref[...] = (part0 + part1).reshape(1, 1, 1)


@jax.jit
def kernel(z, embedding):
    B, D, T = z.shape
    K = embedding.shape[0]

    grid = (B,)
    zq, idx3, loss_raw = pl.pallas_call(
        _vq_block_kernel,
        grid=grid,
        in_specs=[
            pl.BlockSpec((1, D, T), lambda b: (b, 0, 0)),
            pl.BlockSpec((K, D), lambda b: (0, 0)),
            pl.BlockSpec((D, K), lambda b: (0, 0)),
        ],
        out_specs=[
            pl.BlockSpec((1, D, T), lambda b: (b, 0, 0)),
            pl.BlockSpec((1, 1, T), lambda b: (b, 0, 0)),
            pl.BlockSpec((1, 1, 1), lambda b: (b, 0, 0)),
        ],
        out_shape=[
            jax.ShapeDtypeStruct((B, D, T), jnp.float32),
            jax.ShapeDtypeStruct((B, 1, T), jnp.int32),
            jax.ShapeDtypeStruct((B, 1, 1), jnp.float32),
        ],
        scratch_shapes=[pltpu.VMEM((K, 1), jnp.float32)],
        compiler_params=pltpu.CompilerParams(
            dimension_semantics=("arbitrary",),
        ),
    )(z, embedding * (-2.0), embedding.T.astype(jnp.bfloat16))

    loss = jnp.sum(loss_raw) * (0.25 / (B * T * D))
    indices = idx3.reshape(B, T)
    return zq, loss, indices


# speedup vs baseline: 1.1204x; 1.1204x over previous
import jax
import jax.numpy as jnp
from jax.experimental import pallas as pl
from jax.experimental.pallas import tpu as pltpu


def _vq_block_kernel(zm_ref, emb_ref, embt_ref, zq_ref, idx_ref, loss_ref,
                     e2_ref):
    k_dim = emb_ref.shape[0]
    t_blk = zm_ref.shape[2]
    half = t_blk // 2

    @pl.when(pl.program_id(0) == 0)
    def _init():
        emb0 = emb_ref[...]
        e2_ref[...] = 0.25 * jnp.sum(emb0 * emb0, axis=1, keepdims=True)

    emb = emb_ref[...]
    embt = embt_ref[...]
    e2 = e2_ref[...]
    iota_k = jax.lax.broadcasted_iota(jnp.int32, (k_dim, half), 0)

    def _half(zb):
        scores = jnp.dot(emb, zb, preferred_element_type=jnp.float32)
        dist = e2 + scores
        idx = jnp.argmin(dist, axis=0)
        onehot = (iota_k == idx[None, :]).astype(jnp.bfloat16)
        zq = jnp.dot(embt, onehot,
                     preferred_element_type=jnp.float32)
        resid = zb - zq
        part = jnp.sum(resid * resid, axis=(0, 1), keepdims=True)
        return zq, idx, part

    zq0, idx0, part0 = _half(zm_ref[0, :, :half])
    zq1, idx1, part1 = _half(zm_ref[0, :, half:])

    zq_ref[0, :, :half] = zq0
    zq_ref[0, :, half:] = zq1
    idx_ref[0, 0, :half] = idx0
    idx_ref[0, 0, half:] = idx1
    loss_ref[...] = (part0 + part1).reshape(1, 1, 1)


@jax.jit
def kernel(z, embedding):
    B, D, T = z.shape
    K = embedding.shape[0]

    grid = (B,)
    zq, idx3, loss_raw = pl.pallas_call(
        _vq_block_kernel,
        grid=grid,
        in_specs=[
            pl.BlockSpec((1, D, T), lambda b: (b, 0, 0)),
            pl.BlockSpec((K, D), lambda b: (0, 0)),
            pl.BlockSpec((D, K), lambda b: (0, 0)),
        ],
        out_specs=[
            pl.BlockSpec((1, D, T), lambda b: (b, 0, 0)),
            pl.BlockSpec((1, 1, T), lambda b: (b, 0, 0)),
            pl.BlockSpec((1, 1, 1), lambda b: (b, 0, 0)),
        ],
        out_shape=[
            jax.ShapeDtypeStruct((B, D, T), jnp.float32),
            jax.ShapeDtypeStruct((B, 1, T), jnp.int32),
            jax.ShapeDtypeStruct((B, 1, 1), jnp.float32),
        ],
        scratch_shapes=[pltpu.VMEM((K, 1), jnp.float32)],
        compiler_params=pltpu.CompilerParams(
            dimension_semantics=("arbitrary",),
        ),
    )(z, embedding * (-2.0), embedding.T.astype(jnp.bfloat16))

    loss = jnp.sum(loss_raw) * (0.25 / (B * T * D))
    indices = idx3.reshape(B, T)
    return zq, loss, indices


# speedup vs baseline: 1.3288x; 1.1860x over previous
import jax
import jax.numpy as jnp
from jax.experimental import pallas as pl
from jax.experimental.pallas import tpu as pltpu


def _vq_block_kernel(zm_ref, emb_ref, zq_ref, idx_ref, loss_ref,
                     emb2_ref, embt_ref, e2_ref):
    k_dim = emb_ref.shape[0]
    t_blk = zm_ref.shape[2]
    half = t_blk // 2
    n_total = zm_ref.shape[1] * t_blk * pl.num_programs(0)

    @pl.when(pl.program_id(0) == 0)
    def _init():
        emb0 = emb_ref[...]
        emb2_ref[...] = -2.0 * emb0
        embt_ref[...] = emb0.T.astype(jnp.bfloat16)
        e2_ref[...] = jnp.sum(emb0 * emb0, axis=1, keepdims=True)
        loss_ref[...] = jnp.zeros((1, 1), jnp.float32)

    emb = emb2_ref[...]
    embt = embt_ref[...]
    e2 = e2_ref[...]
    iota_k = jax.lax.broadcasted_iota(jnp.int32, (k_dim, half), 0)

    def _half(zb):
        scores = jnp.dot(emb, zb, preferred_element_type=jnp.float32)
        dist = e2 + scores
        idx = jnp.argmin(dist, axis=0)
        onehot = (iota_k == idx[None, :]).astype(jnp.bfloat16)
        zq = jnp.dot(embt, onehot,
                     preferred_element_type=jnp.float32)
        resid = zb - zq
        part = jnp.sum(resid * resid, axis=(0, 1), keepdims=True)
        return zq, idx, part

    zq0, idx0, part0 = _half(zm_ref[0, :, :half])
    zq1, idx1, part1 = _half(zm_ref[0, :, half:])

    zq_ref[0, :, :half] = zq0
    zq_ref[0, :, half:] = zq1
    idx_ref[0, 0, :half] = idx0
    idx_ref[0, 0, half:] = idx1
    loss_ref[...] += part0 + part1

    @pl.when(pl.program_id(0) == pl.num_programs(0) - 1)
    def _fin():
        loss_ref[...] = loss_ref[...] * (0.25 / n_total)


@jax.jit
def kernel(z, embedding):
    B, D, T = z.shape
    K = embedding.shape[0]

    grid = (B,)
    zq, idx3, loss_out = pl.pallas_call(
        _vq_block_kernel,
        grid=grid,
        in_specs=[
            pl.BlockSpec((1, D, T), lambda b: (b, 0, 0)),
            pl.BlockSpec((K, D), lambda b: (0, 0)),
        ],
        out_specs=[
            pl.BlockSpec((1, D, T), lambda b: (b, 0, 0)),
            pl.BlockSpec((1, 1, T), lambda b: (b, 0, 0)),
            pl.BlockSpec((1, 1), lambda b: (0, 0)),
        ],
        out_shape=[
            jax.ShapeDtypeStruct((B, D, T), jnp.float32),
            jax.ShapeDtypeStruct((B, 1, T), jnp.int32),
            jax.ShapeDtypeStruct((1, 1), jnp.float32),
        ],
        scratch_shapes=[
            pltpu.VMEM((K, D), jnp.float32),
            pltpu.VMEM((D, K), jnp.bfloat16),
            pltpu.VMEM((K, 1), jnp.float32),
        ],
        compiler_params=pltpu.CompilerParams(
            dimension_semantics=("arbitrary",),
        ),
    )(z, embedding)

    return zq, loss_out[0, 0], idx3.reshape(B, T)


# speedup vs baseline: 1.4333x; 1.0787x over previous
import jax
import jax.numpy as jnp
from jax.experimental import pallas as pl
from jax.experimental.pallas import tpu as pltpu


def _vq_block_kernel(zm_ref, emb_ref, zq_ref, idx_ref, loss_ref,
                     emb2_ref, embt_ref, e2_ref):
    k_dim = emb_ref.shape[0]
    t_blk = zm_ref.shape[2]
    half = t_blk // 4
    n_total = zm_ref.shape[1] * t_blk * pl.num_programs(0)

    @pl.when(pl.program_id(0) == 0)
    def _init():
        emb0 = emb_ref[...]
        emb2_ref[...] = -2.0 * emb0
        embt_ref[...] = emb0.T.astype(jnp.bfloat16)
        e2_ref[...] = jnp.sum(emb0 * emb0, axis=1, keepdims=True)
        loss_ref[...] = jnp.zeros((1, 1), jnp.float32)

    emb = emb2_ref[...]
    embt = embt_ref[...]
    e2 = e2_ref[...]
    iota_k = jax.lax.broadcasted_iota(jnp.int32, (k_dim, half), 0)

    def _half(zb):
        scores = jnp.dot(emb, zb, preferred_element_type=jnp.float32)
        dist = e2 + scores
        idx = jnp.argmin(dist, axis=0)
        onehot = (iota_k == idx[None, :]).astype(jnp.bfloat16)
        zq = jnp.dot(embt, onehot,
                     preferred_element_type=jnp.float32)
        resid = zb - zq
        part = jnp.sum(resid * resid, axis=(0, 1), keepdims=True)
        return zq, idx, part

    acc = None
    for q in range(4):
        sl = pl.ds(q * half, half)
        zq_q, idx_q, part_q = _half(zm_ref[0, :, sl])
        zq_ref[0, :, sl] = zq_q
        idx_ref[0, 0, sl] = idx_q
        acc = part_q if acc is None else acc + part_q
    loss_ref[...] += acc

    @pl.when(pl.program_id(0) == pl.num_programs(0) - 1)
    def _fin():
        loss_ref[...] = loss_ref[...] * (0.25 / n_total)


@jax.jit
def kernel(z, embedding):
    B, D, T = z.shape
    K = embedding.shape[0]

    grid = (B,)
    zq, idx3, loss_out = pl.pallas_call(
        _vq_block_kernel,
        grid=grid,
        in_specs=[
            pl.BlockSpec((1, D, T), lambda b: (b, 0, 0)),
            pl.BlockSpec((K, D), lambda b: (0, 0)),
        ],
        out_specs=[
            pl.BlockSpec((1, D, T), lambda b: (b, 0, 0)),
            pl.BlockSpec((1, 1, T), lambda b: (b, 0, 0)),
            pl.BlockSpec((1, 1), lambda b: (0, 0)),
        ],
        out_shape=[
            jax.ShapeDtypeStruct((B, D, T), jnp.float32),
            jax.ShapeDtypeStruct((B, 1, T), jnp.int32),
            jax.ShapeDtypeStruct((1, 1), jnp.float32),
        ],
        scratch_shapes=[
            pltpu.VMEM((K, D), jnp.float32),
            pltpu.VMEM((D, K), jnp.bfloat16),
            pltpu.VMEM((K, 1), jnp.float32),
        ],
        compiler_params=pltpu.CompilerParams(
            dimension_semantics=("arbitrary",),
        ),
    )(z, embedding)

    return zq, loss_out[0, 0], idx3.reshape(B, T)
